# Initial kernel scaffold; baseline (speedup 1.0000x reference)
import functools

import jax
import jax.numpy as jnp
from jax import lax
from jax.experimental import pallas as pl
from jax.experimental.pallas import tpu as pltpu
from jax.experimental.pallas import tpu_sc as plsc

N_USERS = 585
N_NODES = 673
D_FEAT = 128
NPAD = 768
FLAT = NPAD * NPAD
SENTINEL = FLAT - 1

NC = 1
NS = 16
NTILES = NC * NS
CHUNK = 128
NCHUNK = 5
K = CHUNK * NCHUNK
E2P = NTILES * K
ZS = FLAT // NS

_sc_mesh = plsc.VectorSubcoreMesh(
    core_axis_name="c", subcore_axis_name="s", num_cores=NC, num_subcores=NS
)


@functools.partial(
    pl.kernel,
    out_type=jax.ShapeDtypeStruct((FLAT,), jnp.float32),
    mesh=_sc_mesh,
    scratch_types=[
        pltpu.VMEM((NCHUNK, CHUNK), jnp.int32),
        pltpu.VMEM((NCHUNK, CHUNK), jnp.float32),
        pltpu.VMEM((ZS,), jnp.float32),
        pltpu.VMEM_SHARED((FLAT,), jnp.float32),
    ],
)
def _sc_build_adj(idx_hbm, out_hbm, idx_v, ones_v, zbuf_v, acc_sh):
    s = lax.axis_index("s")
    wid = s * NC + lax.axis_index("c")

    def fill_ones(i, carry):
        for j in range(NCHUNK):
            ones_v[j, pl.ds(i * 16, 16)] = jnp.ones((16,), jnp.float32)
        return carry

    lax.fori_loop(0, CHUNK // 16, fill_ones, 0)

    def fill_zeros(i, carry):
        for j in range(8):
            zbuf_v[pl.ds((i * 8 + j) * 16, 16)] = jnp.zeros((16,), jnp.float32)
        return carry

    lax.fori_loop(0, ZS // (16 * 8), fill_zeros, 0)

    pltpu.sync_copy(idx_hbm.at[wid], idx_v)
    pltpu.sync_copy(zbuf_v, acc_sh.at[pl.ds(s * ZS, ZS)])
    plsc.subcore_barrier()

    for j in range(NCHUNK):
        pltpu.sync_copy(ones_v.at[j], acc_sh.at[idx_v.at[j]], add=True)
    plsc.subcore_barrier()

    pltpu.sync_copy(acc_sh.at[pl.ds(s * ZS, ZS)], out_hbm.at[pl.ds(s * ZS, ZS)])


def _tc_gcn_body(adj_ref, x_ref, out_ref):
    a = adj_ref[...]
    deg_r = jnp.sum(a, axis=1, keepdims=True)
    deg_c = jnp.sum(a, axis=0, keepdims=True)
    scale = lax.rsqrt(deg_r + 1e-8) * lax.rsqrt(deg_c + 1e-8)
    an = a * scale

    def mm(m, v):
        return lax.dot_general(
            m, v, (((1,), (0,)), ((), ())),
            preferred_element_type=jnp.float32,
            precision=lax.Precision.HIGHEST,
        )

    x0 = x_ref[...]
    x1 = mm(an, x0)
    x2 = mm(an, x1)
    x3 = mm(an, x2)
    out_ref[...] = 0.25 * (x0 + x1 + x2 + x3)


_tc_gcn = pl.pallas_call(
    _tc_gcn_body,
    out_shape=jax.ShapeDtypeStruct((NPAD, D_FEAT), jnp.float32),
)


def kernel(features, edge_src, edge_dst):
    src = edge_src.astype(jnp.int32)
    dst = edge_dst.astype(jnp.int32) + N_USERS
    flat = jnp.concatenate([src * NPAD + dst, dst * NPAD + src])
    flat = jnp.concatenate(
        [flat, jnp.full((E2P - flat.shape[0],), SENTINEL, jnp.int32)]
    )
    adj_flat = _sc_build_adj(flat.reshape(NTILES, NCHUNK, CHUNK))
    adj = adj_flat.reshape(NPAD, NPAD)
    xpad = jnp.pad(features, ((0, NPAD - N_NODES), (0, 0)))
    out = _tc_gcn(adj, xpad)
    return out[:N_NODES]


# speedup vs baseline: 9.6804x; 9.6804x over previous
import functools

import jax
import jax.numpy as jnp
from jax import lax
from jax.experimental import pallas as pl
from jax.experimental.pallas import tpu as pltpu
from jax.experimental.pallas import tpu_sc as plsc

N_USERS = 585
N_NODES = 673
D_FEAT = 128
NPAD = 768
FLAT = NPAD * NPAD
SENTINEL = FLAT - 1

NC = 1
NS = 16
NTILES = NC * NS
CHUNK = 128
NCHUNK = 5
K = CHUNK * NCHUNK
E2P = NTILES * K
ZS = FLAT // NS

def _sc_build_adj_body(idx_hbm, out_hbm, idx_v, ones_v, zbuf_v, acc_sh):
    s = lax.axis_index("s")
    wid = s * NC + lax.axis_index("c")

    def fill_ones(i, carry):
        for j in range(NCHUNK):
            ones_v[j, pl.ds(i * 16, 16)] = jnp.ones((16,), jnp.float32)
        return carry

    lax.fori_loop(0, CHUNK // 16, fill_ones, 0)

    def fill_zeros(i, carry):
        for j in range(8):
            zbuf_v[pl.ds((i * 8 + j) * 16, 16)] = jnp.zeros((16,), jnp.float32)
        return carry

    lax.fori_loop(0, ZS // (16 * 8), fill_zeros, 0)

    pltpu.sync_copy(idx_hbm.at[wid], idx_v)
    pltpu.sync_copy(zbuf_v, acc_sh.at[pl.ds(s * ZS, ZS)])
    plsc.subcore_barrier()

    for j in range(NCHUNK):
        pltpu.sync_copy(ones_v.at[j], acc_sh.at[idx_v.at[j]], add=True)
    plsc.subcore_barrier()

    pltpu.sync_copy(acc_sh.at[pl.ds(s * ZS, ZS)], out_hbm.at[pl.ds(s * ZS, ZS)])


@functools.cache
def _get_sc_build_adj():
    mesh = plsc.VectorSubcoreMesh(
        core_axis_name="c", subcore_axis_name="s", num_cores=NC, num_subcores=NS
    )
    return pl.kernel(
        _sc_build_adj_body,
        out_type=jax.ShapeDtypeStruct((FLAT,), jnp.float32),
        mesh=mesh,
        scratch_types=[
            pltpu.VMEM((NCHUNK, CHUNK), jnp.int32),
            pltpu.VMEM((NCHUNK, CHUNK), jnp.float32),
            pltpu.VMEM((ZS,), jnp.float32),
            pltpu.VMEM_SHARED((FLAT,), jnp.float32),
        ],
    )


def _tc_gcn_body(adj_ref, x_ref, out_ref):
    a = adj_ref[...]
    deg_r = jnp.sum(a, axis=1, keepdims=True)
    deg_c = jnp.sum(a, axis=0, keepdims=True)
    scale = lax.rsqrt(deg_r + 1e-8) * lax.rsqrt(deg_c + 1e-8)
    an = a * scale

    def mm(m, v):
        return lax.dot_general(
            m, v, (((1,), (0,)), ((), ())),
            preferred_element_type=jnp.float32,
            precision=lax.Precision.HIGHEST,
        )

    x0 = x_ref[...]
    x1 = mm(an, x0)
    x2 = mm(an, x1)
    x3 = mm(an, x2)
    out_ref[...] = 0.25 * (x0 + x1 + x2 + x3)


_tc_gcn = pl.pallas_call(
    _tc_gcn_body,
    out_shape=jax.ShapeDtypeStruct((NPAD, D_FEAT), jnp.float32),
)


def kernel(features, edge_src, edge_dst):
    src = edge_src.astype(jnp.int32)
    dst = edge_dst.astype(jnp.int32) + N_USERS
    flat = jnp.concatenate([src * NPAD + dst, dst * NPAD + src])
    flat = jnp.concatenate(
        [flat, jnp.full((E2P - flat.shape[0],), SENTINEL, jnp.int32)]
    )
    adj_flat = _get_sc_build_adj()(flat.reshape(NTILES, NCHUNK, CHUNK))
    adj = adj_flat.reshape(NPAD, NPAD)
    xpad = jnp.pad(features, ((0, NPAD - N_NODES), (0, 0)))
    out = _tc_gcn(adj, xpad)
    return out[:N_NODES]


# speedup vs baseline: 15.3362x; 1.5843x over previous
import functools

import jax
import jax.numpy as jnp
from jax import lax
from jax.experimental import pallas as pl
from jax.experimental.pallas import tpu as pltpu
from jax.experimental.pallas import tpu_sc as plsc

N_USERS = 585
N_ITEMS = 88
N_NODES = 673
D_FEAT = 128
NU_PAD = 592
NI_PAD = 128
FLAT = NU_PAD * NI_PAD
SENTINEL = FLAT - 1

NC = 1
NS = 16
NTILES = NC * NS
CHUNK = 112
NCHUNK = 3
K = CHUNK * NCHUNK
E_PAD = NTILES * K
ZS = FLAT // NS


def _sc_build_adj_body(idx_hbm, out_hbm, idx_v, ones_v, zbuf_v, acc_sh):
    s = lax.axis_index("s")
    wid = s * NC + lax.axis_index("c")

    for j in range(NCHUNK):
        for i in range(CHUNK // 16):
            ones_v[j, pl.ds(i * 16, 16)] = jnp.ones((16,), jnp.float32)

    def fill_zeros(i, carry):
        for j in range(8):
            zbuf_v[pl.ds((i * 8 + j) * 16, 16)] = jnp.zeros((16,), jnp.float32)
        return carry

    lax.fori_loop(0, ZS // (16 * 8), fill_zeros, 0)

    pltpu.sync_copy(idx_hbm.at[wid], idx_v)
    pltpu.sync_copy(zbuf_v, acc_sh.at[pl.ds(s * ZS, ZS)])
    plsc.subcore_barrier()

    for j in range(NCHUNK):
        pltpu.sync_copy(ones_v.at[j], acc_sh.at[idx_v.at[j]], add=True)
    plsc.subcore_barrier()

    pltpu.sync_copy(acc_sh.at[pl.ds(s * ZS, ZS)], out_hbm.at[pl.ds(s * ZS, ZS)])


@functools.cache
def _get_sc_build_adj():
    mesh = plsc.VectorSubcoreMesh(
        core_axis_name="c", subcore_axis_name="s", num_cores=NC, num_subcores=NS
    )
    return pl.kernel(
        _sc_build_adj_body,
        out_type=jax.ShapeDtypeStruct((FLAT,), jnp.float32),
        mesh=mesh,
        scratch_types=[
            pltpu.VMEM((NCHUNK, CHUNK), jnp.int32),
            pltpu.VMEM((NCHUNK, CHUNK), jnp.float32),
            pltpu.VMEM((ZS,), jnp.float32),
            pltpu.VMEM_SHARED((FLAT,), jnp.float32),
        ],
    )


def _tc_gcn_body(b_ref, xu_ref, xi_ref, ou_ref, oi_ref):
    b = b_ref[...]
    du = jnp.sum(b, axis=1, keepdims=True)
    di = jnp.sum(b, axis=0, keepdims=True)
    bn = b * (lax.rsqrt(du + 1e-8) * lax.rsqrt(di + 1e-8))

    def mm(m, v):
        return lax.dot_general(
            m, v, (((1,), (0,)), ((), ())),
            preferred_element_type=jnp.float32,
            precision=lax.Precision.HIGHEST,
        )

    def mmT(m, v):
        return lax.dot_general(
            m, v, (((0,), (0,)), ((), ())),
            preferred_element_type=jnp.float32,
            precision=lax.Precision.HIGHEST,
        )

    xu0, xi0 = xu_ref[...], xi_ref[...]
    xu1, xi1 = mm(bn, xi0), mmT(bn, xu0)
    xu2, xi2 = mm(bn, xi1), mmT(bn, xu1)
    xu3, xi3 = mm(bn, xi2), mmT(bn, xu2)
    ou_ref[...] = 0.25 * (xu0 + xu1 + xu2 + xu3)
    oi_ref[...] = 0.25 * (xi0 + xi1 + xi2 + xi3)


_tc_gcn = pl.pallas_call(
    _tc_gcn_body,
    out_shape=(
        jax.ShapeDtypeStruct((NU_PAD, D_FEAT), jnp.float32),
        jax.ShapeDtypeStruct((NI_PAD, D_FEAT), jnp.float32),
    ),
)


def kernel(features, edge_src, edge_dst):
    src = edge_src.astype(jnp.int32)
    dst = edge_dst.astype(jnp.int32)
    flat = src * NI_PAD + dst
    flat = jnp.concatenate(
        [flat, jnp.full((E_PAD - flat.shape[0],), SENTINEL, jnp.int32)]
    )
    adj_flat = _get_sc_build_adj()(flat.reshape(NTILES, NCHUNK, CHUNK))
    b = adj_flat.reshape(NU_PAD, NI_PAD)
    xu = jnp.pad(features[:N_USERS], ((0, NU_PAD - N_USERS), (0, 0)))
    xi = jnp.pad(features[N_USERS:], ((0, NI_PAD - N_ITEMS), (0, 0)))
    ou, oi = _tc_gcn(b, xu, xi)
    return jnp.concatenate([ou[:N_USERS], oi[:N_ITEMS]])


# speedup vs baseline: 16.6475x; 1.0855x over previous
import functools

import jax
import jax.numpy as jnp
from jax import lax
from jax.experimental import pallas as pl
from jax.experimental.pallas import tpu as pltpu
from jax.experimental.pallas import tpu_sc as plsc

N_USERS = 585
N_ITEMS = 88
N_NODES = 673
N_EDGES = 4680
D_FEAT = 128
NU_PAD = 592
NI_PAD = 128
FLAT = NU_PAD * NI_PAD
SENTINEL = FLAT - 1

NC = 1
NS = 16
NWORK = 15
EPT = N_EDGES // NWORK
CHUNK = 80
NCHUNK = 4
ZS = FLAT // NS


def _sc_build_adj_body(src_hbm, dst_hbm, out_hbm,
                       src_v, dst_v, idx_v, ones_v, zbuf_v, acc_sh):
    s = lax.axis_index("s")
    wid = s * NC + lax.axis_index("c")

    for j in range(NCHUNK):
        for i in range(CHUNK // 16):
            ones_v[j, pl.ds(i * 16, 16)] = jnp.ones((16,), jnp.float32)

    def fill_zeros(i, carry):
        for j in range(8):
            zbuf_v[pl.ds((i * 8 + j) * 16, 16)] = jnp.zeros((16,), jnp.float32)
        return carry

    lax.fori_loop(0, ZS // (16 * 8), fill_zeros, 0)

    @pl.when(wid < NWORK)
    def _stage_and_index():
        pltpu.sync_copy(src_hbm.at[pl.ds(wid * EPT, EPT)],
                        src_v.at[pl.ds(0, EPT)])
        pltpu.sync_copy(dst_hbm.at[pl.ds(wid * EPT, EPT)],
                        dst_v.at[pl.ds(0, EPT)])
        lanes = lax.iota(jnp.int32, 16)
        for j in range(NCHUNK):
            for i in range(CHUNK // 16):
                base = j * CHUNK + i * 16
                f = src_v[pl.ds(base, 16)] * NI_PAD + dst_v[pl.ds(base, 16)] + 1
                if base + 16 > EPT:
                    f = jnp.where(lanes < EPT - base, f, SENTINEL)
                idx_v[j, pl.ds(i * 16, 16)] = f

    pltpu.sync_copy(zbuf_v, acc_sh.at[pl.ds(s * ZS, ZS)])
    plsc.subcore_barrier()

    @pl.when(wid < NWORK)
    def _scatter():
        for j in range(NCHUNK):
            pltpu.sync_copy(ones_v.at[j], acc_sh.at[idx_v.at[j]], add=True)

    plsc.subcore_barrier()

    pltpu.sync_copy(acc_sh.at[pl.ds(s * ZS, ZS)], out_hbm.at[pl.ds(s * ZS, ZS)])


@functools.cache
def _get_sc_build_adj():
    mesh = plsc.VectorSubcoreMesh(
        core_axis_name="c", subcore_axis_name="s", num_cores=NC, num_subcores=NS
    )
    nslot = NCHUNK * CHUNK
    return pl.kernel(
        _sc_build_adj_body,
        out_type=jax.ShapeDtypeStruct((FLAT,), jnp.float32),
        mesh=mesh,
        scratch_types=[
            pltpu.VMEM((nslot,), jnp.int32),
            pltpu.VMEM((nslot,), jnp.int32),
            pltpu.VMEM((NCHUNK, CHUNK), jnp.int32),
            pltpu.VMEM((NCHUNK, CHUNK), jnp.float32),
            pltpu.VMEM((ZS,), jnp.float32),
            pltpu.VMEM_SHARED((FLAT,), jnp.float32),
        ],
    )


def _tc_gcn_body(b_ref, x_ref, out_ref):
    b = b_ref[...]
    du = jnp.sum(b, axis=1, keepdims=True)
    di = jnp.sum(b, axis=0, keepdims=True)
    bn = b * (lax.rsqrt(du + 1e-8) * lax.rsqrt(di + 1e-8))

    x = x_ref[...]
    xu0 = x[0:NU_PAD]
    xi0 = jnp.concatenate(
        [x[584:N_NODES], jnp.zeros((NI_PAD - (N_NODES - 584), D_FEAT),
                                   jnp.float32)], axis=0)

    def mm(m, v):
        return lax.dot_general(
            m, v, (((1,), (0,)), ((), ())),
            preferred_element_type=jnp.float32,
            precision=lax.Precision.HIGHEST,
        )

    def mmT(m, v):
        return lax.dot_general(
            m, v, (((0,), (0,)), ((), ())),
            preferred_element_type=jnp.float32,
            precision=lax.Precision.HIGHEST,
        )

    xu1, xi1 = mm(bn, xi0), mmT(bn, xu0)
    xu2, xi2 = mm(bn, xi1), mmT(bn, xu1)
    xu3, xi3 = mm(bn, xi2), mmT(bn, xu2)
    ou = 0.25 * (xu0 + xu1 + xu2 + xu3)
    oi = 0.25 * (xi0 + xi1 + xi2 + xi3)
    out_ref[...] = jnp.concatenate(
        [ou[0:N_USERS], oi[1:1 + N_ITEMS]], axis=0)


_tc_gcn = pl.pallas_call(
    _tc_gcn_body,
    out_shape=jax.ShapeDtypeStruct((N_NODES, D_FEAT), jnp.float32),
)


def kernel(features, edge_src, edge_dst):
    b_flat = _get_sc_build_adj()(edge_src.astype(jnp.int32),
                                 edge_dst.astype(jnp.int32))
    return _tc_gcn(b_flat.reshape(NU_PAD, NI_PAD), features)
